# Initial kernel scaffold; baseline (speedup 1.0000x reference)
import jax
import jax.numpy as jnp
from jax.experimental import pallas as pl


def kernel(e_logits, a_logits, node_filter_mask, targets, loss_mask, weight):
    raise NotImplementedError("write your pallas kernel here")


# speedup vs baseline: 1.8255x; 1.8255x over previous
import jax
import jax.numpy as jnp
from jax import lax
from jax.experimental import pallas as pl
from jax.experimental.pallas import tpu as pltpu

_B = 128
_N = 32768
_TOP_K = 5
_ROWS = 8
_NEG = -3.0e38


def _fused_body(e_ref, a_ref, mask_ref, tgt_ref, lm_ref, w_ref,
                logits_ref, stats_ref, pred_ref):
    w0 = w_ref[0]
    w1 = w_ref[1]
    e = e_ref[...]
    a = a_ref[...]
    e_max = jnp.max(e, axis=1, keepdims=True)
    a_max = jnp.max(a, axis=1, keepdims=True)
    e_sh = e - e_max
    a_sh = a - a_max
    e_lse = jnp.log(jnp.sum(jnp.exp(e_sh), axis=1, keepdims=True))
    a_lse = jnp.log(jnp.sum(jnp.exp(a_sh), axis=1, keepdims=True))
    bias = jnp.where(mask_ref[...] > 0, 0.0, -999999.0).astype(jnp.float32)
    logits = (e_sh - e_lse) * w0 + (a_sh - a_lse) * w1 + bias
    logits_ref[...] = logits

    m = jnp.max(logits, axis=1, keepdims=True)
    logz = jnp.log(jnp.sum(jnp.exp(logits - m), axis=1, keepdims=True)) + m

    iota = lax.broadcasted_iota(jnp.int32, (_ROWS, _N), 1)
    tgt = tgt_ref[...][:, 0:1]
    tgt_logit = jnp.sum(jnp.where(iota == tgt, logits, 0.0), axis=1,
                        keepdims=True)
    lm = lm_ref[...][:, 0:1]
    losses = (logz - tgt_logit) * lm

    col = lax.broadcasted_iota(jnp.int32, (_ROWS, 128), 1)
    stats_ref[...] = jnp.where(col == 0, losses, 0.0)

    cur = logits
    pred = jnp.zeros((_ROWS, 128), jnp.int32)
    for i in range(_TOP_K):
        mi = jnp.max(cur, axis=1, keepdims=True)
        idx = jnp.min(jnp.where(cur == mi, iota, _N), axis=1, keepdims=True)
        pred = jnp.where(col == i, idx, pred)
        cur = jnp.where(iota == idx, _NEG, cur)
    pred_ref[...] = pred


def kernel(e_logits, a_logits, node_filter_mask, targets, loss_mask, weight):
    mask2d = node_filter_mask.reshape(1, _N)
    tgt2d = jnp.broadcast_to(targets[:, None], (_B, 128))
    lm2d = jnp.broadcast_to(loss_mask[:, None], (_B, 128))
    logits, stats, pred = pl.pallas_call(
        _fused_body,
        grid=(_B // _ROWS,),
        in_specs=[
            pl.BlockSpec((_ROWS, _N), lambda i: (i, 0)),
            pl.BlockSpec((_ROWS, _N), lambda i: (i, 0)),
            pl.BlockSpec((1, _N), lambda i: (0, 0)),
            pl.BlockSpec((_ROWS, 128), lambda i: (i, 0)),
            pl.BlockSpec((_ROWS, 128), lambda i: (i, 0)),
            pl.BlockSpec(memory_space=pltpu.SMEM),
        ],
        out_specs=[
            pl.BlockSpec((_ROWS, _N), lambda i: (i, 0)),
            pl.BlockSpec((_ROWS, 128), lambda i: (i, 0)),
            pl.BlockSpec((_ROWS, 128), lambda i: (i, 0)),
        ],
        out_shape=[
            jax.ShapeDtypeStruct((_B, _N), jnp.float32),
            jax.ShapeDtypeStruct((_B, 128), jnp.float32),
            jax.ShapeDtypeStruct((_B, 128), jnp.int32),
        ],
    )(e_logits, a_logits, mask2d, tgt2d, lm2d, weight)
    return logits, stats[:, 0], pred[:, :_TOP_K]


# speedup vs baseline: 1.9713x; 1.0799x over previous
import jax
import jax.numpy as jnp
from jax import lax
from jax.experimental import pallas as pl
from jax.experimental.pallas import tpu as pltpu

_B = 128
_N = 32768
_TOP_K = 5
_ROWS = 8
_NEG = -3.0e38


def _fused_body(e_ref, a_ref, mask_ref, tgt_ref, lm_ref, w_ref,
                logits_ref, stats_ref, pred_ref):
    w0 = w_ref[0]
    w1 = w_ref[1]
    e = e_ref[...]
    a = a_ref[...]
    e_max = jnp.max(e, axis=1, keepdims=True)
    a_max = jnp.max(a, axis=1, keepdims=True)
    e_lse = jnp.log(jnp.sum(jnp.exp(e - e_max), axis=1, keepdims=True))
    a_lse = jnp.log(jnp.sum(jnp.exp(a - a_max), axis=1, keepdims=True))
    c = (e_max + e_lse) * w0 + (a_max + a_lse) * w1
    bias = jnp.where(mask_ref[...] > 0, 0.0, -999999.0).astype(jnp.float32)
    logits = e * w0 + a * w1 + (bias - c)
    logits_ref[...] = logits

    m = jnp.max(logits, axis=1, keepdims=True)
    logz = jnp.log(jnp.sum(jnp.exp(logits - m), axis=1, keepdims=True)) + m

    iota = lax.broadcasted_iota(jnp.int32, (_ROWS, _N), 1)
    tgt = tgt_ref[...][:, 0:1]
    tgt_logit = jnp.sum(jnp.where(iota == tgt, logits, 0.0), axis=1,
                        keepdims=True)
    lm = lm_ref[...][:, 0:1]
    losses = (logz - tgt_logit) * lm

    col = lax.broadcasted_iota(jnp.int32, (_ROWS, 128), 1)
    stats_ref[...] = jnp.where(col == 0, losses, 0.0)

    cur = logits
    pred = jnp.zeros((_ROWS, 128), jnp.int32)
    for i in range(_TOP_K):
        mi = jnp.max(cur, axis=1, keepdims=True)
        idx = jnp.min(jnp.where(cur == mi, iota, _N), axis=1, keepdims=True)
        pred = jnp.where(col == i, idx, pred)
        cur = jnp.where(iota == idx, _NEG, cur)
    pred_ref[...] = pred


def kernel(e_logits, a_logits, node_filter_mask, targets, loss_mask, weight):
    mask2d = node_filter_mask.reshape(1, _N)
    tgt2d = jnp.broadcast_to(targets[:, None], (_B, 128))
    lm2d = jnp.broadcast_to(loss_mask[:, None], (_B, 128))
    logits, stats, pred = pl.pallas_call(
        _fused_body,
        grid=(_B // _ROWS,),
        in_specs=[
            pl.BlockSpec((_ROWS, _N), lambda i: (i, 0)),
            pl.BlockSpec((_ROWS, _N), lambda i: (i, 0)),
            pl.BlockSpec((1, _N), lambda i: (0, 0)),
            pl.BlockSpec((_ROWS, 128), lambda i: (i, 0)),
            pl.BlockSpec((_ROWS, 128), lambda i: (i, 0)),
            pl.BlockSpec(memory_space=pltpu.SMEM),
        ],
        out_specs=[
            pl.BlockSpec((_ROWS, _N), lambda i: (i, 0)),
            pl.BlockSpec((_ROWS, 128), lambda i: (i, 0)),
            pl.BlockSpec((_ROWS, 128), lambda i: (i, 0)),
        ],
        out_shape=[
            jax.ShapeDtypeStruct((_B, _N), jnp.float32),
            jax.ShapeDtypeStruct((_B, 128), jnp.float32),
            jax.ShapeDtypeStruct((_B, 128), jnp.int32),
        ],
    )(e_logits, a_logits, mask2d, tgt2d, lm2d, weight)
    return logits, stats[:, 0], pred[:, :_TOP_K]
